# Initial kernel scaffold; baseline (speedup 1.0000x reference)
import jax
import jax.numpy as jnp
from jax.experimental import pallas as pl


def kernel(edge_attr, edge_index, W, b):
    raise NotImplementedError("write your pallas kernel here")


# speedup vs baseline: 23.1278x; 23.1278x over previous
import functools

import jax
import jax.numpy as jnp
from jax import lax
from jax.experimental import pallas as pl
from jax.experimental.pallas import tpu as pltpu
from jax.experimental.pallas import tpu_sc as plsc

N_NODES = 100000
N_PAD = 102400
NC, NS = 2, 16
NW = NC * NS
E_TOTAL = 3200000
EPW = E_TOTAL // NW
CH = 4000
L = 16

_SC_MESH = plsc.VectorSubcoreMesh(
    core_axis_name="c", subcore_axis_name="s", num_cores=NC, num_subcores=NS
)


def _k1_body(a_ref, w_ref, v_ref, m_ref):
    i = pl.program_id(0)
    v = jnp.dot(a_ref[...], w_ref[...],
                preferred_element_type=jnp.float32)[:, 0]
    v_ref[...] = v
    bm = jnp.max(v)

    @pl.when(i == 0)
    def _():
        m_ref[0] = bm

    @pl.when(i > 0)
    def _():
        m_ref[0] = jnp.maximum(m_ref[0], bm)


def _k1(edge_attr, W):
    E, D = edge_attr.shape
    BE = 25600
    grid = E // BE
    return pl.pallas_call(
        _k1_body,
        grid=(grid,),
        in_specs=[
            pl.BlockSpec((BE, D), lambda i: (i, 0)),
            pl.BlockSpec((D, 1), lambda i: (0, 0)),
        ],
        out_specs=[
            pl.BlockSpec((BE,), lambda i: (i,)),
            pl.BlockSpec(memory_space=pltpu.SMEM),
        ],
        out_shape=[
            jax.ShapeDtypeStruct((E,), jnp.float32),
            jax.ShapeDtypeStruct((1,), jnp.float32),
        ],
    )(edge_attr, W)


def _k2_body(v_hbm, g_hbm, m_hbm, s32_hbm, s_tab, v_ch, g_ch, m_v):
    cid = lax.axis_index("c")
    sid = lax.axis_index("s")
    wid = cid * NS + sid
    base = wid * EPW

    pltpu.sync_copy(m_hbm, m_v)
    mv = m_v[...]

    zero = jnp.zeros((L,), jnp.float32)

    def zloop(i, _):
        s_tab[pl.ds(i * L, L)] = zero
        return 0

    lax.fori_loop(0, N_PAD // L, zloop, 0)

    def chunk_loop(k, _):
        off = base + k * CH
        pltpu.sync_copy(v_hbm.at[pl.ds(off, CH)], v_ch)
        pltpu.sync_copy(g_hbm.at[pl.ds(off, CH)], g_ch)

        def inner(j, _):
            vv = v_ch[pl.ds(j * L, L)]
            ss = g_ch[pl.ds(j * L, L)]
            e = jnp.exp(vv - mv)
            plsc.addupdate_scatter(s_tab, [ss], e)
            return 0

        lax.fori_loop(0, CH // L, inner, 0)
        return 0

    lax.fori_loop(0, EPW // CH, chunk_loop, 0)
    pltpu.sync_copy(s_tab, s32_hbm.at[wid])


_k2 = functools.partial(
    pl.kernel,
    _k2_body,
    out_type=jax.ShapeDtypeStruct((NW, N_PAD), jnp.float32),
    mesh=_SC_MESH,
    compiler_params=pltpu.CompilerParams(needs_layout_passes=False),
    scratch_types=[
        pltpu.VMEM((N_PAD,), jnp.float32),
        pltpu.VMEM((CH,), jnp.float32),
        pltpu.VMEM((CH,), jnp.int32),
        pltpu.VMEM((L,), jnp.float32),
    ],
)()


def _k3_body(sp_ref, s_ref):
    s_ref[...] = jnp.sum(sp_ref[...], axis=0)


def _k3(s_parts):
    NB = N_PAD // 4
    return pl.pallas_call(
        _k3_body,
        grid=(N_PAD // NB,),
        in_specs=[pl.BlockSpec((NW, NB), lambda i: (0, i))],
        out_specs=pl.BlockSpec((NB,), lambda i: (i,)),
        out_shape=jax.ShapeDtypeStruct((N_PAD,), jnp.float32),
    )(s_parts)


def _k4_body(v_hbm, g_hbm, m_hbm, s_hbm, o_hbm, s_tab, v_ch, g_ch, o_ch, m_v):
    cid = lax.axis_index("c")
    sid = lax.axis_index("s")
    wid = cid * NS + sid
    base = wid * EPW

    pltpu.sync_copy(m_hbm, m_v)
    mv = m_v[...]
    pltpu.sync_copy(s_hbm, s_tab)

    def chunk_loop(k, _):
        off = base + k * CH
        pltpu.sync_copy(v_hbm.at[pl.ds(off, CH)], v_ch)
        pltpu.sync_copy(g_hbm.at[pl.ds(off, CH)], g_ch)

        def inner(j, _):
            vv = v_ch[pl.ds(j * L, L)]
            ss = g_ch[pl.ds(j * L, L)]
            e = jnp.exp(vv - mv)
            sv = plsc.load_gather(s_tab, [ss])
            o_ch[pl.ds(j * L, L)] = e / sv
            return 0

        lax.fori_loop(0, CH // L, inner, 0)
        pltpu.sync_copy(o_ch, o_hbm.at[pl.ds(off, CH)])
        return 0

    lax.fori_loop(0, EPW // CH, chunk_loop, 0)


_k4 = functools.partial(
    pl.kernel,
    _k4_body,
    out_type=jax.ShapeDtypeStruct((E_TOTAL,), jnp.float32),
    mesh=_SC_MESH,
    compiler_params=pltpu.CompilerParams(needs_layout_passes=False),
    scratch_types=[
        pltpu.VMEM((N_PAD,), jnp.float32),
        pltpu.VMEM((CH,), jnp.float32),
        pltpu.VMEM((CH,), jnp.int32),
        pltpu.VMEM((CH,), jnp.float32),
        pltpu.VMEM((L,), jnp.float32),
    ],
)()


def kernel(edge_attr, edge_index, W, b):
    seg = edge_index[0]
    v, m = _k1(edge_attr, W)
    m16 = jnp.broadcast_to(m, (L,))
    s_parts = _k2(v, seg, m16)
    s = _k3(s_parts)
    out = _k4(v, seg, m16, s)
    return out[:, None]


# speedup vs baseline: 35.4601x; 1.5332x over previous
import functools

import jax
import jax.numpy as jnp
from jax import lax
from jax.experimental import pallas as pl
from jax.experimental.pallas import tpu as pltpu
from jax.experimental.pallas import tpu_sc as plsc

N_NODES = 100000
N_PAD = 102400
NC, NS = 2, 16
NW = NC * NS
E_TOTAL = 3200000
EPW = E_TOTAL // NW
CH = 4000
L = 16

_SC_MESH = plsc.VectorSubcoreMesh(
    core_axis_name="c", subcore_axis_name="s", num_cores=NC, num_subcores=NS
)


def _k1_body(a_ref, m2_ref, v_ref, m_ref):
    i = pl.program_id(0)
    v = jnp.dot(a_ref[...], m2_ref[...], preferred_element_type=jnp.float32)
    v_ref[...] = v
    bm = jnp.max(v)

    @pl.when(i == 0)
    def _():
        m_ref[0] = bm

    @pl.when(i > 0)
    def _():
        m_ref[0] = jnp.maximum(m_ref[0], bm)


def _k1(a2, M2):
    R = a2.shape[0]
    BR = 1000
    grid = R // BR
    return pl.pallas_call(
        _k1_body,
        grid=(grid,),
        in_specs=[
            pl.BlockSpec((BR, 2048), lambda i: (i, 0)),
            pl.BlockSpec((2048, 128), lambda i: (0, 0)),
        ],
        out_specs=[
            pl.BlockSpec((BR, 128), lambda i: (i, 0)),
            pl.BlockSpec(memory_space=pltpu.SMEM),
        ],
        out_shape=[
            jax.ShapeDtypeStruct((R, 128), jnp.float32),
            jax.ShapeDtypeStruct((1,), jnp.float32),
        ],
    )(a2, M2)


def _k2_body(v_hbm, g_hbm, m_hbm, s32_hbm, s_tab, v_ch, g_ch, m_v):
    cid = lax.axis_index("c")
    sid = lax.axis_index("s")
    wid = cid * NS + sid
    base = wid * EPW

    pltpu.sync_copy(m_hbm, m_v)
    mv = m_v[...]

    zero = jnp.zeros((L,), jnp.float32)

    def zloop(i, _):
        s_tab[pl.ds(i * L, L)] = zero
        return 0

    lax.fori_loop(0, N_PAD // L, zloop, 0)

    def chunk_loop(k, _):
        off = base + k * CH
        pltpu.sync_copy(v_hbm.at[pl.ds(off, CH)], v_ch)
        pltpu.sync_copy(g_hbm.at[pl.ds(off, CH)], g_ch)

        def inner(j, _):
            vv = v_ch[pl.ds(j * L, L)]
            ss = g_ch[pl.ds(j * L, L)]
            e = jnp.exp(vv - mv)
            plsc.addupdate_scatter(s_tab, [ss], e)
            return 0

        lax.fori_loop(0, CH // L, inner, 0)
        return 0

    lax.fori_loop(0, EPW // CH, chunk_loop, 0)
    pltpu.sync_copy(s_tab, s32_hbm.at[wid])


_k2 = functools.partial(
    pl.kernel,
    _k2_body,
    out_type=jax.ShapeDtypeStruct((NW, N_PAD), jnp.float32),
    mesh=_SC_MESH,
    compiler_params=pltpu.CompilerParams(needs_layout_passes=False),
    scratch_types=[
        pltpu.VMEM((N_PAD,), jnp.float32),
        pltpu.VMEM((CH,), jnp.float32),
        pltpu.VMEM((CH,), jnp.int32),
        pltpu.VMEM((L,), jnp.float32),
    ],
)()


def _k3_body(sp_ref, s_ref):
    s_ref[...] = jnp.sum(sp_ref[...], axis=0)


def _k3(s_parts):
    NB = N_PAD // 4
    return pl.pallas_call(
        _k3_body,
        grid=(N_PAD // NB,),
        in_specs=[pl.BlockSpec((NW, NB), lambda i: (0, i))],
        out_specs=pl.BlockSpec((NB,), lambda i: (i,)),
        out_shape=jax.ShapeDtypeStruct((N_PAD,), jnp.float32),
    )(s_parts)


def _k4_body(v_hbm, g_hbm, m_hbm, s_hbm, o_hbm, s_tab, v_ch, g_ch, o_ch, m_v):
    cid = lax.axis_index("c")
    sid = lax.axis_index("s")
    wid = cid * NS + sid
    base = wid * EPW

    pltpu.sync_copy(m_hbm, m_v)
    mv = m_v[...]
    pltpu.sync_copy(s_hbm, s_tab)

    def chunk_loop(k, _):
        off = base + k * CH
        pltpu.sync_copy(v_hbm.at[pl.ds(off, CH)], v_ch)
        pltpu.sync_copy(g_hbm.at[pl.ds(off, CH)], g_ch)

        def inner(j, _):
            vv = v_ch[pl.ds(j * L, L)]
            ss = g_ch[pl.ds(j * L, L)]
            e = jnp.exp(vv - mv)
            sv = plsc.load_gather(s_tab, [ss])
            o_ch[pl.ds(j * L, L)] = e / sv
            return 0

        lax.fori_loop(0, CH // L, inner, 0)
        pltpu.sync_copy(o_ch, o_hbm.at[pl.ds(off, CH)])
        return 0

    lax.fori_loop(0, EPW // CH, chunk_loop, 0)


_k4 = functools.partial(
    pl.kernel,
    _k4_body,
    out_type=jax.ShapeDtypeStruct((E_TOTAL,), jnp.float32),
    mesh=_SC_MESH,
    compiler_params=pltpu.CompilerParams(needs_layout_passes=False),
    scratch_types=[
        pltpu.VMEM((N_PAD,), jnp.float32),
        pltpu.VMEM((CH,), jnp.float32),
        pltpu.VMEM((CH,), jnp.int32),
        pltpu.VMEM((CH,), jnp.float32),
        pltpu.VMEM((L,), jnp.float32),
    ],
)()


def kernel(edge_attr, edge_index, W, b):
    seg = edge_index[0]
    eye16 = jnp.eye(16, dtype=jnp.float32)
    eye8 = jnp.eye(8, dtype=jnp.float32)
    M2 = jnp.einsum("d,rs,ct->rcdst", W[:, 0], eye16, eye8).reshape(2048, 128)
    a2 = edge_attr.reshape(E_TOTAL * 16 // 2048, 2048)
    v2, m = _k1(a2, M2)
    v = v2.reshape(E_TOTAL)
    m16 = jnp.broadcast_to(m, (L,))
    s_parts = _k2(v, seg, m16)
    s = _k3(s_parts)
    out = _k4(v, seg, m16, s)
    return out[:, None]


# speedup vs baseline: 168.5645x; 4.7536x over previous
import functools

import jax
import jax.numpy as jnp
from jax import lax
from jax.experimental import pallas as pl
from jax.experimental.pallas import tpu as pltpu
from jax.experimental.pallas import tpu_sc as plsc

N_NODES = 100000
N_PAD = 102400
NC, NS = 2, 16
NW = NC * NS
E_TOTAL = 3200000
EPW = E_TOTAL // NW
CH = 4000
L = 16

_SC_MESH = plsc.VectorSubcoreMesh(
    core_axis_name="c", subcore_axis_name="s", num_cores=NC, num_subcores=NS
)


def _k1_body(a_ref, w_ref, v_ref, m_ref):
    i = pl.program_id(0)
    v = jnp.sum(a_ref[...] * w_ref[...], axis=0)
    v_ref[...] = v
    bm = jnp.max(v)

    @pl.when(i == 0)
    def _():
        m_ref[0] = bm

    @pl.when(i > 0)
    def _():
        m_ref[0] = jnp.maximum(m_ref[0], bm)


def _k1(eaT, W):
    D, E = eaT.shape
    BE = 128000
    grid = E // BE
    return pl.pallas_call(
        _k1_body,
        grid=(grid,),
        in_specs=[
            pl.BlockSpec((D, BE), lambda i: (0, i)),
            pl.BlockSpec((D, 1), lambda i: (0, 0)),
        ],
        out_specs=[
            pl.BlockSpec((BE,), lambda i: (i,)),
            pl.BlockSpec(memory_space=pltpu.SMEM),
        ],
        out_shape=[
            jax.ShapeDtypeStruct((E,), jnp.float32),
            jax.ShapeDtypeStruct((1,), jnp.float32),
        ],
    )(eaT, W)


def _k2_body(v_hbm, g_hbm, m_hbm, s32_hbm, s_tab, v_ch, g_ch, m_v):
    cid = lax.axis_index("c")
    sid = lax.axis_index("s")
    wid = cid * NS + sid
    base = wid * EPW

    pltpu.sync_copy(m_hbm, m_v)
    mv = m_v[...]

    zero = jnp.zeros((L,), jnp.float32)

    def zloop(i, _):
        s_tab[pl.ds(i * L, L)] = zero
        return 0

    lax.fori_loop(0, N_PAD // L, zloop, 0)

    def chunk_loop(k, _):
        off = base + k * CH
        pltpu.sync_copy(v_hbm.at[pl.ds(off, CH)], v_ch)
        pltpu.sync_copy(g_hbm.at[pl.ds(off, CH)], g_ch)

        def inner(j, _):
            vv = v_ch[pl.ds(j * L, L)]
            ss = g_ch[pl.ds(j * L, L)]
            e = jnp.exp(vv - mv)
            plsc.addupdate_scatter(s_tab, [ss], e)
            return 0

        lax.fori_loop(0, CH // L, inner, 0)
        return 0

    lax.fori_loop(0, EPW // CH, chunk_loop, 0)
    pltpu.sync_copy(s_tab, s32_hbm.at[wid])


_k2 = functools.partial(
    pl.kernel,
    _k2_body,
    out_type=jax.ShapeDtypeStruct((NW, N_PAD), jnp.float32),
    mesh=_SC_MESH,
    compiler_params=pltpu.CompilerParams(needs_layout_passes=False),
    scratch_types=[
        pltpu.VMEM((N_PAD,), jnp.float32),
        pltpu.VMEM((CH,), jnp.float32),
        pltpu.VMEM((CH,), jnp.int32),
        pltpu.VMEM((L,), jnp.float32),
    ],
)()


def _k3_body(sp_ref, s_ref):
    s_ref[...] = jnp.sum(sp_ref[...], axis=0)


def _k3(s_parts):
    NB = N_PAD // 4
    return pl.pallas_call(
        _k3_body,
        grid=(N_PAD // NB,),
        in_specs=[pl.BlockSpec((NW, NB), lambda i: (0, i))],
        out_specs=pl.BlockSpec((NB,), lambda i: (i,)),
        out_shape=jax.ShapeDtypeStruct((N_PAD,), jnp.float32),
    )(s_parts)


def _k4_body(v_hbm, g_hbm, m_hbm, s_hbm, o_hbm, s_tab, v_ch, g_ch, o_ch, m_v):
    cid = lax.axis_index("c")
    sid = lax.axis_index("s")
    wid = cid * NS + sid
    base = wid * EPW

    pltpu.sync_copy(m_hbm, m_v)
    mv = m_v[...]
    pltpu.sync_copy(s_hbm, s_tab)

    def chunk_loop(k, _):
        off = base + k * CH
        pltpu.sync_copy(v_hbm.at[pl.ds(off, CH)], v_ch)
        pltpu.sync_copy(g_hbm.at[pl.ds(off, CH)], g_ch)

        def inner(j, _):
            vv = v_ch[pl.ds(j * L, L)]
            ss = g_ch[pl.ds(j * L, L)]
            e = jnp.exp(vv - mv)
            sv = plsc.load_gather(s_tab, [ss])
            o_ch[pl.ds(j * L, L)] = e / sv
            return 0

        lax.fori_loop(0, CH // L, inner, 0)
        pltpu.sync_copy(o_ch, o_hbm.at[pl.ds(off, CH)])
        return 0

    lax.fori_loop(0, EPW // CH, chunk_loop, 0)


_k4 = functools.partial(
    pl.kernel,
    _k4_body,
    out_type=jax.ShapeDtypeStruct((E_TOTAL,), jnp.float32),
    mesh=_SC_MESH,
    compiler_params=pltpu.CompilerParams(needs_layout_passes=False),
    scratch_types=[
        pltpu.VMEM((N_PAD,), jnp.float32),
        pltpu.VMEM((CH,), jnp.float32),
        pltpu.VMEM((CH,), jnp.int32),
        pltpu.VMEM((CH,), jnp.float32),
        pltpu.VMEM((L,), jnp.float32),
    ],
)()


def kernel(edge_attr, edge_index, W, b):
    seg = edge_index[0]
    v, m = _k1(edge_attr.T, W)
    m16 = jnp.broadcast_to(m, (L,))
    s_parts = _k2(v, seg, m16)
    s = _k3(s_parts)
    out = _k4(v, seg, m16, s)
    return out[:, None]


# speedup vs baseline: 265.1331x; 1.5729x over previous
import functools

import jax
import jax.numpy as jnp
from jax import lax
from jax.experimental import pallas as pl
from jax.experimental.pallas import tpu as pltpu
from jax.experimental.pallas import tpu_sc as plsc

N_NODES = 100000
N_PAD = 102400
NC, NS = 2, 16
NW = NC * NS
E_TOTAL = 3200000
EPW = E_TOTAL // NW
CH = 2000
NCH = EPW // CH
L = 16
UNROLL = 5

_SC_MESH = plsc.VectorSubcoreMesh(
    core_axis_name="c", subcore_axis_name="s", num_cores=NC, num_subcores=NS
)


def _k1_body(a_ref, w_ref, v_ref, m_ref):
    i = pl.program_id(0)
    v = jnp.sum(a_ref[...] * w_ref[...], axis=0)
    v_ref[...] = v
    bm = jnp.max(v)

    @pl.when(i == 0)
    def _():
        m_ref[0] = bm

    @pl.when(i > 0)
    def _():
        m_ref[0] = jnp.maximum(m_ref[0], bm)


def _k1(eaT, W):
    D, E = eaT.shape
    BE = 128000
    grid = E // BE
    return pl.pallas_call(
        _k1_body,
        grid=(grid,),
        in_specs=[
            pl.BlockSpec((D, BE), lambda i: (0, i)),
            pl.BlockSpec((D, 1), lambda i: (0, 0)),
        ],
        out_specs=[
            pl.BlockSpec((BE,), lambda i: (i,)),
            pl.BlockSpec(memory_space=pltpu.SMEM),
        ],
        out_shape=[
            jax.ShapeDtypeStruct((E,), jnp.float32),
            jax.ShapeDtypeStruct((1,), jnp.float32),
        ],
    )(eaT, W)


def _k2_body(v_hbm, g_hbm, m_hbm, s32_hbm, s_tab, v_ch0, v_ch1, g_ch0, g_ch1,
             m_v, sems):
    cid = lax.axis_index("c")
    sid = lax.axis_index("s")
    wid = cid * NS + sid
    base = wid * EPW
    v_chs, g_chs = (v_ch0, v_ch1), (g_ch0, g_ch1)

    def start(k, slot):
        off = base + k * CH
        pltpu.async_copy(v_hbm.at[pl.ds(off, CH)], v_chs[slot], sems.at[slot])
        pltpu.async_copy(g_hbm.at[pl.ds(off, CH)], g_chs[slot],
                         sems.at[2 + slot])

    def wait(k, slot):
        off = base + k * CH
        pltpu.make_async_copy(v_hbm.at[pl.ds(off, CH)], v_chs[slot],
                              sems.at[slot]).wait()
        pltpu.make_async_copy(g_hbm.at[pl.ds(off, CH)], g_chs[slot],
                              sems.at[2 + slot]).wait()

    def compute(slot):
        def inner(jj, _):
            for u in range(UNROLL):
                j = jj * UNROLL + u
                vv = v_chs[slot][pl.ds(j * L, L)]
                ss = g_chs[slot][pl.ds(j * L, L)]
                e = jnp.exp(vv - mv)
                plsc.addupdate_scatter(s_tab, [ss], e)
            return 0

        lax.fori_loop(0, CH // L // UNROLL, inner, 0)

    start(0, 0)
    pltpu.sync_copy(m_hbm, m_v)
    mv = m_v[...]

    zero = jnp.zeros((L,), jnp.float32)

    def zloop(i, _):
        for u in range(8):
            s_tab[pl.ds((i * 8 + u) * L, L)] = zero
        return 0

    lax.fori_loop(0, N_PAD // L // 8, zloop, 0)

    def pair_loop(k2, _):
        k0 = 2 * k2
        start(k0 + 1, 1)
        wait(k0, 0)
        compute(0)

        @pl.when(k2 < NCH // 2 - 1)
        def _():
            start(k0 + 2, 0)

        wait(k0 + 1, 1)
        compute(1)
        return 0

    lax.fori_loop(0, NCH // 2, pair_loop, 0)
    pltpu.sync_copy(s_tab, s32_hbm.at[wid])


_k2 = functools.partial(
    pl.kernel,
    _k2_body,
    out_type=jax.ShapeDtypeStruct((NW, N_PAD), jnp.float32),
    mesh=_SC_MESH,
    compiler_params=pltpu.CompilerParams(needs_layout_passes=False),
    scratch_types=[
        pltpu.VMEM((N_PAD,), jnp.float32),
        pltpu.VMEM((CH,), jnp.float32),
        pltpu.VMEM((CH,), jnp.float32),
        pltpu.VMEM((CH,), jnp.int32),
        pltpu.VMEM((CH,), jnp.int32),
        pltpu.VMEM((L,), jnp.float32),
        pltpu.SemaphoreType.DMA((4,)),
    ],
)()


def _k3_body(sp_ref, s_ref):
    s_ref[...] = 1.0 / jnp.sum(sp_ref[...], axis=0)


def _k3(s_parts):
    NB = N_PAD // 4
    return pl.pallas_call(
        _k3_body,
        grid=(N_PAD // NB,),
        in_specs=[pl.BlockSpec((NW, NB), lambda i: (0, i))],
        out_specs=pl.BlockSpec((NB,), lambda i: (i,)),
        out_shape=jax.ShapeDtypeStruct((N_PAD,), jnp.float32),
    )(s_parts)


def _k4_body(v_hbm, g_hbm, m_hbm, r_hbm, o_hbm,
             r_tab, v_ch0, v_ch1, g_ch0, g_ch1, o_ch0, o_ch1, m_v, sems):
    cid = lax.axis_index("c")
    sid = lax.axis_index("s")
    wid = cid * NS + sid
    base = wid * EPW
    v_chs, g_chs, o_chs = (v_ch0, v_ch1), (g_ch0, g_ch1), (o_ch0, o_ch1)

    def start(k, slot):
        off = base + k * CH
        pltpu.async_copy(v_hbm.at[pl.ds(off, CH)], v_chs[slot], sems.at[slot])
        pltpu.async_copy(g_hbm.at[pl.ds(off, CH)], g_chs[slot],
                         sems.at[2 + slot])

    def wait(k, slot):
        off = base + k * CH
        pltpu.make_async_copy(v_hbm.at[pl.ds(off, CH)], v_chs[slot],
                              sems.at[slot]).wait()
        pltpu.make_async_copy(g_hbm.at[pl.ds(off, CH)], g_chs[slot],
                              sems.at[2 + slot]).wait()

    def store_start(k, slot):
        off = base + k * CH
        pltpu.async_copy(o_chs[slot], o_hbm.at[pl.ds(off, CH)],
                         sems.at[4 + slot])

    def store_wait(k, slot):
        off = base + k * CH
        pltpu.make_async_copy(o_chs[slot], o_hbm.at[pl.ds(off, CH)],
                              sems.at[4 + slot]).wait()

    def compute(slot):
        def inner(jj, _):
            for u in range(UNROLL):
                j = jj * UNROLL + u
                vv = v_chs[slot][pl.ds(j * L, L)]
                ss = g_chs[slot][pl.ds(j * L, L)]
                e = jnp.exp(vv - mv)
                rv = plsc.load_gather(r_tab, [ss])
                o_chs[slot][pl.ds(j * L, L)] = e * rv
            return 0

        lax.fori_loop(0, CH // L // UNROLL, inner, 0)

    start(0, 0)
    pltpu.sync_copy(m_hbm, m_v)
    mv = m_v[...]
    pltpu.sync_copy(r_hbm, r_tab)

    def pair_loop(k2, _):
        k0 = 2 * k2
        start(k0 + 1, 1)
        wait(k0, 0)

        @pl.when(k2 > 0)
        def _():
            store_wait(k0 - 2, 0)

        compute(0)
        store_start(k0, 0)

        @pl.when(k2 < NCH // 2 - 1)
        def _():
            start(k0 + 2, 0)

        wait(k0 + 1, 1)

        @pl.when(k2 > 0)
        def _():
            store_wait(k0 - 1, 1)

        compute(1)
        store_start(k0 + 1, 1)
        return 0

    lax.fori_loop(0, NCH // 2, pair_loop, 0)
    store_wait(NCH - 2, 0)
    store_wait(NCH - 1, 1)


_k4 = functools.partial(
    pl.kernel,
    _k4_body,
    out_type=jax.ShapeDtypeStruct((E_TOTAL,), jnp.float32),
    mesh=_SC_MESH,
    compiler_params=pltpu.CompilerParams(needs_layout_passes=False),
    scratch_types=[
        pltpu.VMEM((N_PAD,), jnp.float32),
        pltpu.VMEM((CH,), jnp.float32),
        pltpu.VMEM((CH,), jnp.float32),
        pltpu.VMEM((CH,), jnp.int32),
        pltpu.VMEM((CH,), jnp.int32),
        pltpu.VMEM((CH,), jnp.float32),
        pltpu.VMEM((CH,), jnp.float32),
        pltpu.VMEM((L,), jnp.float32),
        pltpu.SemaphoreType.DMA((6,)),
    ],
)()


def kernel(edge_attr, edge_index, W, b):
    seg = edge_index[0]
    v, m = _k1(edge_attr.T, W)
    m16 = jnp.broadcast_to(m, (L,))
    s_parts = _k2(v, seg, m16)
    s = _k3(s_parts)
    out = _k4(v, seg, m16, s)
    return out[:, None]
